# baseline (speedup 1.0000x reference)
import functools

import jax
import jax.numpy as jnp
from jax import lax
from jax.experimental import pallas as pl
from jax.experimental.pallas import tpu as pltpu
from jax.experimental.pallas import tpu_sc as plsc

N, C, H, W = 2, 128, 56, 56
PH = PW = 7
G = 2
SCALE = 0.25
OFFSET = 0.5
K = 1000
NBIN = PH * PW
NPIX = K * NBIN
NPIX_PAD = 49152
NROWS = N * H * W
NCG = 8
CCG = C // NCG
NPG = 4
PER_TEC = NPIX_PAD // NPG
PCH = 256
NCH = PER_TEC // PCH
NJ = 16


def _prep_body(rt_ref, idx_ref, w_ref):
    r = rt_ref[...]
    bk = r[0].astype(jnp.int32)
    sw = r[1] * SCALE - OFFSET
    sh = r[2] * SCALE - OFFSET
    ew = r[3] * SCALE - OFFSET
    eh = r[4] * SCALE - OFFSET
    bin_h = (eh - sh) / PH
    bin_w = (ew - sw) / PW

    pyi = lax.broadcasted_iota(jnp.int32, (PH, G, K), 0).astype(jnp.float32)
    sv = (lax.broadcasted_iota(jnp.int32, (PH, G, K), 1).astype(jnp.float32)
          + 0.5) / G
    y = sh[None, None, :] + (pyi + sv) * bin_h[None, None, :]
    my = ((y >= -1.0) & (y <= H)).astype(jnp.float32)
    yc = jnp.clip(y, 0.0, H - 1)
    y0f = jnp.floor(yc)
    y0 = y0f.astype(jnp.int32)
    ly = yc - y0f
    hy = 1.0 - ly
    y1 = jnp.minimum(y0 + 1, H - 1)

    x = sw[None, None, :] + (pyi + sv) * bin_w[None, None, :]
    mx = ((x >= -1.0) & (x <= W)).astype(jnp.float32)
    xc = jnp.clip(x, 0.0, W - 1)
    x0f = jnp.floor(xc)
    x0 = x0f.astype(jnp.int32)
    lx = xc - x0f
    hx = 1.0 - lx
    x1 = jnp.minimum(x0 + 1, W - 1)

    ybase = bk * (H * W)
    inv = 1.0 / (G * G)
    for j in range(NJ):
        sy, sx, cy, cx = (j >> 3) & 1, (j >> 2) & 1, (j >> 1) & 1, j & 1
        yw = (my * (hy if cy == 0 else ly))[:, sy, :]
        yi = (y0 if cy == 0 else y1)[:, sy, :]
        xw = (mx * (hx if cx == 0 else lx))[:, sx, :]
        xi = (x0 if cx == 0 else x1)[:, sx, :]
        wj = (yw[:, None, :] * xw[None, :, :] * inv).reshape(NBIN, K)
        ij = (ybase[None, None, :] + yi[:, None, :] * W
              + xi[None, :, :]).reshape(NBIN, K)
        idx_ref[j] = ij
        w_ref[j] = wj


def _sc_body(idx_hbm, w_hbm, tab_hbm, out_hbm, tab_v, idx_v, w_v, out_v):
    wid = lax.axis_index("s") * 2 + lax.axis_index("c")
    pg = wid % NPG
    cg = wid // NPG
    pltpu.sync_copy(tab_hbm.at[cg], tab_v)
    base = pg * PER_TEC

    def chunk(ci, carry):
        qb = base + ci * PCH
        pltpu.sync_copy(idx_hbm.at[pl.ds(qb, PCH)], idx_v)
        pltpu.sync_copy(w_hbm.at[pl.ds(qb, PCH)], w_v)

        def pix(p, c2):
            iv = idx_v[p]
            wv = w_v[p]
            acc = tab_v[iv[0]] * wv[0]
            for j in range(1, NJ):
                acc = acc + tab_v[iv[j]] * wv[j]
            out_v[p] = acc
            return c2

        lax.fori_loop(0, PCH, pix, 0)
        pltpu.sync_copy(out_v, out_hbm.at[cg, pl.ds(qb, PCH)])
        return carry

    lax.fori_loop(0, NCH, chunk, 0)


def kernel(features, rois):
    rt = jnp.transpose(rois, (1, 0))
    idx3, w3 = pl.pallas_call(
        _prep_body,
        out_shape=[
            jax.ShapeDtypeStruct((NJ, NBIN, K), jnp.int32),
            jax.ShapeDtypeStruct((NJ, NBIN, K), jnp.float32),
        ],
    )(rt)
    idxf = jnp.pad(jnp.transpose(idx3.reshape(NJ, NPIX), (1, 0)),
                   ((0, NPIX_PAD - NPIX), (0, 0)))
    wf = jnp.pad(jnp.transpose(w3.reshape(NJ, NPIX), (1, 0)),
                 ((0, NPIX_PAD - NPIX), (0, 0)))

    tab = jnp.transpose(
        jnp.transpose(features, (0, 2, 3, 1)).reshape(NROWS, NCG, CCG),
        (1, 0, 2))

    sc_fn = pl.kernel(
        _sc_body,
        out_type=jax.ShapeDtypeStruct((NCG, NPIX_PAD, CCG), jnp.float32),
        mesh=plsc.VectorSubcoreMesh(core_axis_name="c", subcore_axis_name="s"),
        compiler_params=pltpu.CompilerParams(use_tc_tiling_on_sc=False),
        scratch_types=[
            pltpu.VMEM((NROWS, CCG), jnp.float32),
            pltpu.VMEM((PCH, NJ), jnp.int32),
            pltpu.VMEM((PCH, NJ), jnp.float32),
            pltpu.VMEM((PCH, CCG), jnp.float32),
        ],
    )
    out8 = sc_fn(idxf, wf, tab)

    out = jnp.transpose(
        out8[:, :NPIX, :].reshape(NCG, NBIN, K, CCG),
        (2, 0, 3, 1)).reshape(K, C, PH, PW)
    return out
